# baseline (speedup 1.0000x reference)
import functools

import jax
import jax.numpy as jnp
from jax import lax
from jax.experimental import pallas as pl
from jax.experimental.pallas import tpu as pltpu
from jax.experimental.pallas import tpu_sc as plsc

_PRIMES = (31, 43, 59, 61, 73, 97, 103, 113)
_BUCKETS = 100000
_K = 8
_SLICE = 16
_BATCH, _SEQ = 4096, 50
_N = _BATCH * _SEQ
_NC, _NS, _L = 2, 16, 16
_NW = _NC * _NS
_TPW = _N // _NW
_CB = 128
_NB = _TPW // _CB

_mesh = plsc.VectorSubcoreMesh(core_axis_name="c", subcore_axis_name="s")


@functools.partial(
    pl.kernel,
    out_type=jax.ShapeDtypeStruct((_N, _K * _SLICE), jnp.float32),
    mesh=_mesh,
    compiler_params=pltpu.CompilerParams(use_tc_tiling_on_sc=False),
    scratch_types=[
        pltpu.VMEM((_CB,), jnp.int32),
        pltpu.VMEM((_K, _CB), jnp.int32),
        pltpu.VMEM((_K, _CB, _SLICE), jnp.float32),
        pltpu.VMEM((_CB, _K * _SLICE), jnp.float32),
        pltpu.SemaphoreType.DMA,
        pltpu.SemaphoreType.DMA,
    ],
)
def _emb(ids_hbm, t0, t1, t2, t3, t4, t5, t6, t7, out_hbm,
         ids_v, idx_v, rows_v, oblk_v, gsem, ssem):
    tables = (t0, t1, t2, t3, t4, t5, t6, t7)
    wid = lax.axis_index("s") * _NC + lax.axis_index("c")
    base = wid * _TPW
    inv = jnp.float32(1.0 / _BUCKETS)

    def block(b, carry):
        tb = base + b * _CB
        pltpu.sync_copy(ids_hbm.at[pl.ds(tb, _CB)], ids_v)
        for j in range(_CB // _L):
            x = ids_v[pl.ds(j * _L, _L)]
            xf = x.astype(jnp.float32)
            for k in range(_K):
                p = _PRIMES[k]
                q = (xf * jnp.float32(p) * inv).astype(jnp.int32)
                r = x * p - q * _BUCKETS
                r = jnp.where(r < 0, r + _BUCKETS, r)
                r = jnp.where(r >= _BUCKETS, r - _BUCKETS, r)
                idx_v[k, pl.ds(j * _L, _L)] = r
        gets = [
            pltpu.async_copy(tables[k].at[idx_v.at[k]], rows_v.at[k], gsem)
            for k in range(_K)
        ]
        for c in gets:
            c.wait()
        def xpose(j, c):
            for k in range(_K):
                oblk_v[j, pl.ds(k * _SLICE, _SLICE)] = rows_v[k, j, :]
            return c

        lax.fori_loop(0, _CB, xpose, 0)
        pltpu.async_copy(oblk_v, out_hbm.at[pl.ds(tb, _CB)], ssem).wait()
        return carry

    lax.fori_loop(0, _NB, block, 0)


def kernel(input_ids, table_0, table_1, table_2, table_3, table_4,
           table_5, table_6, table_7):
    ids = input_ids.reshape(-1).astype(jnp.int32)
    out = _emb(ids, table_0, table_1, table_2, table_3,
               table_4, table_5, table_6, table_7)
    return out.reshape(_BATCH, _SEQ, _K * _SLICE)


# speedup vs baseline: 1.3220x; 1.3220x over previous
import functools

import jax
import jax.numpy as jnp
from jax import lax
from jax.experimental import pallas as pl
from jax.experimental.pallas import tpu as pltpu
from jax.experimental.pallas import tpu_sc as plsc

_PRIMES = (31, 43, 59, 61, 73, 97, 103, 113)
_BUCKETS = 100000
_K = 8
_SLICE = 16
_BATCH, _SEQ = 4096, 50
_N = _BATCH * _SEQ
_NC, _NS, _L = 2, 16, 16
_NW = _NC * _NS
_TPW = _N // _NW
_CB = 2 * _SEQ
_NB = _TPW // _CB
_CHUNKS = (0, 16, 32, 48, 64, 80, 84)

_mesh = plsc.VectorSubcoreMesh(core_axis_name="c", subcore_axis_name="s")


@functools.partial(
    pl.kernel,
    out_type=jax.ShapeDtypeStruct((_BATCH, _SEQ, _K * _SLICE), jnp.float32),
    mesh=_mesh,
    compiler_params=pltpu.CompilerParams(use_tc_tiling_on_sc=False),
    scratch_types=[
        pltpu.VMEM((_TPW,), jnp.int32),
        pltpu.VMEM((2, _K, _CB), jnp.int32),
        pltpu.VMEM((2, _K, _CB, _SLICE), jnp.float32),
        pltpu.VMEM((2, 2, _SEQ, _K * _SLICE), jnp.float32),
        pltpu.SemaphoreType.DMA,
        pltpu.SemaphoreType.DMA,
        pltpu.SemaphoreType.DMA,
        pltpu.SemaphoreType.DMA,
    ],
)
def _emb(ids_hbm, t0, t1, t2, t3, t4, t5, t6, t7, out_hbm,
         ids_v, idx_v, rows_v, oblk_v, gsem0, gsem1, idsem, ssem):
    tables = (t0, t1, t2, t3, t4, t5, t6, t7)
    gsems = (gsem0, gsem1)
    wid = lax.axis_index("s") * _NC + lax.axis_index("c")
    base = wid * _TPW
    bat0 = wid * (_TPW // _SEQ)
    inv = jnp.float32(1.0 / _BUCKETS)

    pltpu.async_copy(ids_hbm.at[pl.ds(base, _TPW)], ids_v, idsem).wait()

    def fill(b, ph):
        for off in _CHUNKS:
            x = ids_v[pl.ds(b * _CB + off, _L)]
            xf = x.astype(jnp.float32)
            for k in range(_K):
                p = _PRIMES[k]
                q = (xf * jnp.float32(p) * inv).astype(jnp.int32)
                r = x * p - q * _BUCKETS
                r = jnp.where(r < 0, r + _BUCKETS, r)
                r = jnp.where(r >= _BUCKETS, r - _BUCKETS, r)
                idx_v[ph, k, pl.ds(off, _L)] = r
        return [
            pltpu.async_copy(tables[k].at[idx_v.at[ph, k]],
                             rows_v.at[ph, k], gsems[ph])
            for k in range(_K)
        ]

    def process(b, ph, gets):
        for c in gets:
            c.wait()
        for j in range(_CB):
            for k in range(_K):
                oblk_v[ph, j // _SEQ, j % _SEQ, pl.ds(k * _SLICE, _SLICE)] = (
                    rows_v[ph, k, j, :])
        return pltpu.async_copy(oblk_v.at[ph],
                                out_hbm.at[pl.ds(bat0 + 2 * b, 2)], ssem)

    fill(0, 0)
    fill(1, 1)

    def pair_nocarry(q, carry):
        b0 = 2 * q
        g0q = [pltpu.make_async_copy(tables[k].at[idx_v.at[0, k]],
                                     rows_v.at[0, k], gsem0)
               for k in range(_K)]
        put0 = process(b0, 0, g0q)
        fill(b0 + 2, 0)
        put0.wait()
        g1q = [pltpu.make_async_copy(tables[k].at[idx_v.at[1, k]],
                                     rows_v.at[1, k], gsem1)
               for k in range(_K)]
        put1 = process(b0 + 1, 1, g1q)
        fill(b0 + 3, 1)
        put1.wait()
        return carry

    lax.fori_loop(0, (_NB - 2) // 2, pair_nocarry, 0)

    gl0 = [pltpu.make_async_copy(tables[k].at[idx_v.at[0, k]],
                                 rows_v.at[0, k], gsem0) for k in range(_K)]
    process(_NB - 2, 0, gl0).wait()
    gl1 = [pltpu.make_async_copy(tables[k].at[idx_v.at[1, k]],
                                 rows_v.at[1, k], gsem1) for k in range(_K)]
    process(_NB - 1, 1, gl1).wait()


def kernel(input_ids, table_0, table_1, table_2, table_3, table_4,
           table_5, table_6, table_7):
    ids = input_ids.reshape(-1).astype(jnp.int32)
    return _emb(ids, table_0, table_1, table_2, table_3,
                table_4, table_5, table_6, table_7)


# speedup vs baseline: 1.5816x; 1.1964x over previous
import functools

import jax
import jax.numpy as jnp
from jax import lax
from jax.experimental import pallas as pl
from jax.experimental.pallas import tpu as pltpu
from jax.experimental.pallas import tpu_sc as plsc

_PRIMES = (31, 43, 59, 61, 73, 97, 103, 113)
_BUCKETS = 100000
_K = 8
_SLICE = 16
_BATCH, _SEQ = 4096, 50
_SEQP = 56
_N = _BATCH * _SEQ
_NC, _NS, _L = 2, 16, 16
_NW = _NC * _NS
_TPW = _N // _NW
_CB = 2 * _SEQ
_NB = _TPW // _CB
_CHUNKS = (0, 16, 32, 48, 64, 80, 84)

_mesh = plsc.VectorSubcoreMesh(core_axis_name="c", subcore_axis_name="s")


@functools.partial(
    pl.kernel,
    out_type=jax.ShapeDtypeStruct((_BATCH, _SEQP, _K * _SLICE), jnp.float32),
    mesh=_mesh,
    compiler_params=pltpu.CompilerParams(use_tc_tiling_on_sc=False),
    scratch_types=[
        pltpu.VMEM((_TPW,), jnp.int32),
        pltpu.VMEM((2, _K, _CB), jnp.int32),
        pltpu.VMEM((2, _K, _CB, _SLICE), jnp.float32),
        pltpu.VMEM((2, 2, _SEQ, _K * _SLICE), jnp.float32),
        pltpu.SemaphoreType.DMA,
        pltpu.SemaphoreType.DMA,
        pltpu.SemaphoreType.DMA,
        pltpu.SemaphoreType.DMA,
    ],
)
def _emb(ids_hbm, t0, t1, t2, t3, t4, t5, t6, t7, out_hbm,
         ids_v, idx_v, rows_v, oblk_v, gsem0, gsem1, idsem, ssem):
    tables = (t0, t1, t2, t3, t4, t5, t6, t7)
    gsems = (gsem0, gsem1)
    wid = lax.axis_index("s") * _NC + lax.axis_index("c")
    base = wid * _TPW
    bat0 = wid * (_TPW // _SEQ)
    inv = jnp.float32(1.0 / _BUCKETS)

    pltpu.async_copy(ids_hbm.at[pl.ds(base, _TPW)], ids_v, idsem).wait()

    def fill(b, ph):
        for off in _CHUNKS:
            x = ids_v[pl.ds(b * _CB + off, _L)]
            xf = x.astype(jnp.float32)
            for k in range(_K):
                p = _PRIMES[k]
                q = (xf * jnp.float32(p) * inv).astype(jnp.int32)
                r = x * p - q * _BUCKETS
                r = jnp.where(r < 0, r + _BUCKETS, r)
                r = jnp.where(r >= _BUCKETS, r - _BUCKETS, r)
                idx_v[ph, k, pl.ds(off, _L)] = r
        for k in range(_K):
            pltpu.async_copy(tables[k].at[idx_v.at[ph, k]],
                             rows_v.at[ph, k], gsems[ph])

    def flush(b, ph):
        for k in range(_K):
            pltpu.make_async_copy(tables[k].at[idx_v.at[ph, k]],
                                  rows_v.at[ph, k], gsems[ph]).wait()
        for j in range(_CB):
            for k in range(_K):
                oblk_v[ph, j // _SEQ, j % _SEQ, pl.ds(k * _SLICE, _SLICE)] = (
                    rows_v[ph, k, j, :])
        return pltpu.async_copy(
            oblk_v.at[ph],
            out_hbm.at[pl.ds(bat0 + 2 * b, 2), pl.ds(0, _SEQ)], ssem)

    fill(0, 0)
    fill(1, 1)

    def pair(q, carry):
        b0 = 2 * q
        put0 = flush(b0, 0)
        fill(b0 + 2, 0)
        put0.wait()
        put1 = flush(b0 + 1, 1)
        fill(b0 + 3, 1)
        put1.wait()
        return carry

    lax.fori_loop(0, (_NB - 2) // 2, pair, 0)
    flush(_NB - 2, 0).wait()
    flush(_NB - 1, 1).wait()


def kernel(input_ids, table_0, table_1, table_2, table_3, table_4,
           table_5, table_6, table_7):
    ids = input_ids.reshape(-1).astype(jnp.int32)
    out = _emb(ids, table_0, table_1, table_2, table_3,
               table_4, table_5, table_6, table_7)
    return out[:, :_SEQ, :]


# speedup vs baseline: 1.5863x; 1.0029x over previous
import functools

import jax
import jax.numpy as jnp
from jax import lax
from jax.experimental import pallas as pl
from jax.experimental.pallas import tpu as pltpu
from jax.experimental.pallas import tpu_sc as plsc

_PRIMES = (31, 43, 59, 61, 73, 97, 103, 113)
_BUCKETS = 100000
_K = 8
_SLICE = 16
_BATCH, _SEQ = 4096, 50
_SEQP = 56
_N = _BATCH * _SEQ
_NC, _NS, _L = 2, 16, 16
_NW = _NC * _NS
_TPW = _N // _NW
_CB = 2 * _SEQ
_NB = _TPW // _CB
_CHUNKS = (0, 16, 32, 48, 64, 80, 84)

_mesh = plsc.VectorSubcoreMesh(core_axis_name="c", subcore_axis_name="s")


@functools.partial(
    pl.kernel,
    out_type=jax.ShapeDtypeStruct((_BATCH * _SEQP * _K * _SLICE,),
                                  jnp.float32),
    mesh=_mesh,
    compiler_params=pltpu.CompilerParams(use_tc_tiling_on_sc=False),
    scratch_types=[
        pltpu.VMEM((_TPW,), jnp.int32),
        pltpu.VMEM((2, _K, _CB), jnp.int32),
        pltpu.VMEM((2, _K, _CB, _SLICE), jnp.float32),
        pltpu.VMEM((2, 2, _SEQ * _K * _SLICE), jnp.float32),
        pltpu.SemaphoreType.DMA,
        pltpu.SemaphoreType.DMA,
        pltpu.SemaphoreType.DMA,
        pltpu.SemaphoreType.DMA,
    ],
)
def _emb(ids_hbm, t0, t1, t2, t3, t4, t5, t6, t7, out_hbm,
         ids_v, idx_v, rows_v, oblk_v, gsem0, gsem1, idsem, ssem):
    tables = (t0, t1, t2, t3, t4, t5, t6, t7)
    gsems = (gsem0, gsem1)
    wid = lax.axis_index("s") * _NC + lax.axis_index("c")
    base = wid * _TPW
    bat0 = wid * (_TPW // _SEQ)
    inv = jnp.float32(1.0 / _BUCKETS)

    pltpu.async_copy(ids_hbm.at[pl.ds(base, _TPW)], ids_v, idsem).wait()

    def fill(b, ph):
        for off in _CHUNKS:
            x = ids_v[pl.ds(b * _CB + off, _L)]
            xf = x.astype(jnp.float32)
            for k in range(_K):
                p = _PRIMES[k]
                q = (xf * jnp.float32(p) * inv).astype(jnp.int32)
                r = x * p - q * _BUCKETS
                r = jnp.where(r < 0, r + _BUCKETS, r)
                r = jnp.where(r >= _BUCKETS, r - _BUCKETS, r)
                idx_v[ph, k, pl.ds(off, _L)] = r
        for k in range(_K):
            pltpu.async_copy(tables[k].at[idx_v.at[ph, k]],
                             rows_v.at[ph, k], gsems[ph])

    def flush(b, ph):
        for k in range(_K):
            pltpu.make_async_copy(tables[k].at[idx_v.at[ph, k]],
                                  rows_v.at[ph, k], gsems[ph]).wait()
        for j in range(_CB):
            for k in range(_K):
                oblk_v[ph, j // _SEQ,
                       pl.ds((j % _SEQ) * (_K * _SLICE) + k * _SLICE,
                             _SLICE)] = rows_v[ph, k, j, :]
        row = _SEQP * _K * _SLICE
        return [
            pltpu.async_copy(
                oblk_v.at[ph, i],
                out_hbm.at[pl.ds((bat0 + 2 * b + i) * row,
                                 _SEQ * _K * _SLICE)], ssem)
            for i in range(2)
        ]

    def wait_all(puts):
        for p in puts:
            p.wait()

    fill(0, 0)
    fill(1, 1)

    def pair(q, carry):
        b0 = 2 * q
        put0 = flush(b0, 0)
        fill(b0 + 2, 0)
        wait_all(put0)
        put1 = flush(b0 + 1, 1)
        fill(b0 + 3, 1)
        wait_all(put1)
        return carry

    lax.fori_loop(0, (_NB - 2) // 2, pair, 0)
    wait_all(flush(_NB - 2, 0))
    wait_all(flush(_NB - 1, 1))


def kernel(input_ids, table_0, table_1, table_2, table_3, table_4,
           table_5, table_6, table_7):
    ids = input_ids.reshape(-1).astype(jnp.int32)
    out = _emb(ids, table_0, table_1, table_2, table_3,
               table_4, table_5, table_6, table_7)
    return out.reshape(_BATCH, _SEQP, _K * _SLICE)[:, :_SEQ, :]
